# baseline (speedup 1.0000x reference)
import functools

import jax
import jax.numpy as jnp
from jax import lax
from jax.experimental import pallas as pl
from jax.experimental.pallas import tpu as pltpu
from jax.experimental.pallas import tpu_sc as plsc

F32 = jnp.float32

NC = 2
NS = 16
WIN = 128
BLK = 512
CH = 32
HCH = 16
NBUF = 4
PIECE = 16


def _cdiv(a, b):
    return (a + b - 1) // b


@functools.lru_cache(maxsize=None)
def _sc_hist(n_pad, r2):
    mesh = plsc.VectorSubcoreMesh(core_axis_name="c", subcore_axis_name="s")
    rpw = n_pad // NS
    zrows = rpw // 4
    rh = r2 // 2

    def body(dst_hbm, out_hbm, dst_v, ones_v, zbuf, acc, sem):
        c = lax.axis_index("c")
        s = lax.axis_index("s")
        pltpu.sync_copy(dst_hbm.at[s, pl.ds(c * rh, rh)], dst_v)

        @pl.loop(0, WIN)
        def _(i):
            ones_v[i, pl.ds(0, HCH)] = jnp.ones((HCH,), F32)

        @pl.loop(0, zrows)
        def _(i):
            zbuf[i, pl.ds(0, HCH)] = jnp.zeros((HCH,), F32)

        for q in range(4):
            pltpu.sync_copy(zbuf, acc.at[pl.ds(s * rpw + q * zrows, zrows)])
        plsc.subcore_barrier()

        @pl.loop(0, rh)
        def _(j):
            pltpu.async_copy(ones_v, acc.at[dst_v.at[j]], sem, add=True).wait()

        plsc.subcore_barrier()
        pltpu.sync_copy(acc.at[pl.ds(s * rpw, rpw)],
                        out_hbm.at[c, pl.ds(s * rpw, rpw)])

    return pl.kernel(
        body,
        mesh=mesh,
        compiler_params=pltpu.CompilerParams(use_tc_tiling_on_sc=False),
        out_type=jax.ShapeDtypeStruct((NC, n_pad, HCH), F32),
        scratch_types=[
            pltpu.VMEM((rh, WIN), jnp.int32),
            pltpu.VMEM((WIN, HCH), F32),
            pltpu.VMEM((zrows, HCH), F32),
            pltpu.VMEM_SHARED((n_pad, HCH), F32),
            pltpu.SemaphoreType.DMA,
        ],
    )


@functools.lru_cache(maxsize=None)
def _sc_scatter(n_chunks, n_pad, r2):
    mesh = plsc.VectorSubcoreMesh(core_axis_name="c", subcore_axis_name="s")
    rpw = n_pad // NS
    zrows = rpw // 32
    cpc = n_chunks // NC
    n_pieces = r2 // PIECE
    assert r2 % PIECE == 0 and PIECE % NBUF == 0

    def body(src8_hbm, dst_hbm, t8_hbm, out_hbm, *rest):
        srcs = rest[0:2]
        dsts = rest[2:4]
        bufs = rest[4:4 + NBUF]
        zbuf = rest[4 + NBUF]
        acc = rest[5 + NBUF]
        sems_g = rest[6 + NBUF:6 + 2 * NBUF]
        sems_s = rest[6 + 2 * NBUF:6 + 3 * NBUF]
        sems_i = rest[6 + 3 * NBUF:8 + 3 * NBUF]
        c = lax.axis_index("c")
        s = lax.axis_index("s")

        @pl.loop(0, zrows)
        def _(i):
            zbuf[i, pl.ds(0, CH)] = jnp.zeros((CH,), F32)

        def load_piece(ci, piece, par):
            pltpu.async_copy(src8_hbm.at[ci, s, pl.ds(piece * PIECE, PIECE)],
                             srcs[par], sems_i[par])
            pltpu.async_copy(dst_hbm.at[s, pl.ds(piece * PIECE, PIECE)],
                             dsts[par], sems_i[par])

        def wait_piece(ci, piece, par):
            pltpu.make_async_copy(src8_hbm.at[ci, s, pl.ds(piece * PIECE, PIECE)],
                                  srcs[par], sems_i[par]).wait()
            pltpu.make_async_copy(dst_hbm.at[s, pl.ds(piece * PIECE, PIECE)],
                                  dsts[par], sems_i[par]).wait()

        for ci_l in range(cpc):
            ci = c * cpc + ci_l
            load_piece(ci, 0, 0)
            for q in range(32):
                pltpu.sync_copy(zbuf, acc.at[pl.ds(s * rpw + q * zrows, zrows)])
            plsc.subcore_barrier()

            for piece in range(n_pieces):
                par = piece % 2
                src_v, dst_v = srcs[par], dsts[par]
                wait_piece(ci, piece, par)
                if piece + 1 < n_pieces:
                    load_piece(ci, piece + 1, 1 - par)

                @pl.loop(0, PIECE, step=NBUF)
                def _(j):
                    @pl.when(j > 0)
                    def _():
                        for b in range(NBUF):
                            pltpu.make_async_copy(
                                bufs[b], acc.at[dst_v.at[j + b]],
                                sems_s[b]).wait()
                    gathers = [
                        pltpu.async_copy(t8_hbm.at[src_v.at[j + b]], bufs[b],
                                         sems_g[b])
                        for b in range(NBUF)
                    ]
                    for b in range(NBUF):
                        gathers[b].wait()
                        pltpu.async_copy(bufs[b], acc.at[dst_v.at[j + b]],
                                         sems_s[b], add=True)

                for b in range(NBUF):
                    pltpu.make_async_copy(
                        bufs[b], acc.at[dst_v.at[PIECE - NBUF + b]],
                        sems_s[b]).wait()

            plsc.subcore_barrier()
            pltpu.sync_copy(acc.at[pl.ds(s * rpw, rpw)],
                            out_hbm.at[pl.ds(s * rpw, rpw),
                                       pl.ds(ci * CH, CH)])
            if ci_l + 1 < cpc:
                plsc.subcore_barrier()

    return pl.kernel(
        body,
        mesh=mesh,
        compiler_params=pltpu.CompilerParams(use_tc_tiling_on_sc=False),
        out_type=jax.ShapeDtypeStruct((n_pad, n_chunks * CH), F32),
        scratch_types=[
            pltpu.VMEM((PIECE, WIN), jnp.int32),
            pltpu.VMEM((PIECE, WIN), jnp.int32),
            pltpu.VMEM((PIECE, WIN), jnp.int32),
            pltpu.VMEM((PIECE, WIN), jnp.int32),
        ] + [pltpu.VMEM((WIN, CH), F32)] * NBUF + [
            pltpu.VMEM((zrows, CH), F32),
            pltpu.VMEM_SHARED((n_pad, CH), F32),
        ] + [pltpu.SemaphoreType.DMA] * (2 * NBUF + 2),
    )


def _dot(a, b):
    return lax.dot_general(a, b, (((1,), (0,)), ((), ())),
                           preferred_element_type=F32)


def _tc_prep(hist, x_p, n_real):
    n_pad, cin = x_p.shape
    nb = n_pad // BLK

    def body(hist_ref, x_ref, dis_ref, t_ref):
        i = pl.program_id(0)
        deg = hist_ref[0] + hist_ref[1] + 1.0
        row = i * BLK + lax.broadcasted_iota(jnp.int32, (BLK, HCH), 0)
        dis = jnp.where(row < n_real, lax.rsqrt(deg), 0.0)
        dis_col = dis[:, 0:1]
        dis_ref[...] = dis_col
        t_ref[...] = x_ref[...] * dis_col

    return pl.pallas_call(
        body,
        grid=(nb,),
        in_specs=[
            pl.BlockSpec((NC, BLK, HCH), lambda i: (0, i, 0)),
            pl.BlockSpec((BLK, cin), lambda i: (i, 0)),
        ],
        out_specs=[pl.BlockSpec((BLK, 1), lambda i: (i, 0)),
                   pl.BlockSpec((BLK, cin), lambda i: (i, 0))],
        out_shape=[jax.ShapeDtypeStruct((n_pad, 1), F32),
                   jax.ShapeDtypeStruct((n_pad, cin), F32)],
    )(hist, x_p)


def _tc_layer(s_full, t_prev, dis, b, W_next):
    n_pad, fin = t_prev.shape
    fout = W_next.shape[1]
    nb = n_pad // BLK

    def body(s_ref, t_ref, dis_ref, b_ref, w_ref, out_ref):
        dis = dis_ref[...]
        h = jnp.maximum(dis * (s_ref[...] + t_ref[...]) + b_ref[...], 0.0)
        out_ref[...] = dis * _dot(h, w_ref[...])

    return pl.pallas_call(
        body,
        grid=(nb,),
        in_specs=[
            pl.BlockSpec((BLK, fin), lambda i: (i, 0)),
            pl.BlockSpec((BLK, fin), lambda i: (i, 0)),
            pl.BlockSpec((BLK, 1), lambda i: (i, 0)),
            pl.BlockSpec((1, fin), lambda i: (0, 0)),
            pl.BlockSpec((fin, fout), lambda i: (0, 0)),
        ],
        out_specs=pl.BlockSpec((BLK, fout), lambda i: (i, 0)),
        out_shape=jax.ShapeDtypeStruct((n_pad, fout), F32),
    )(s_full, t_prev, dis, b.reshape(1, -1), W_next)


def _tc_layer1(s_full, t0, dis, W1, b1, W2):
    n_pad, cin = t0.shape
    hid = W1.shape[1]
    mid = W2.shape[1]
    nb = n_pad // BLK

    def body(s_ref, t_ref, dis_ref, w1_ref, b1_ref, w2_ref, out_ref):
        dis = dis_ref[...]
        p0 = dis * (s_ref[...] + t_ref[...])
        h1 = jnp.maximum(_dot(p0, w1_ref[...]) + b1_ref[...], 0.0)
        out_ref[...] = dis * _dot(h1, w2_ref[...])

    return pl.pallas_call(
        body,
        grid=(nb,),
        in_specs=[
            pl.BlockSpec((BLK, cin), lambda i: (i, 0)),
            pl.BlockSpec((BLK, cin), lambda i: (i, 0)),
            pl.BlockSpec((BLK, 1), lambda i: (i, 0)),
            pl.BlockSpec((cin, hid), lambda i: (0, 0)),
            pl.BlockSpec((1, hid), lambda i: (0, 0)),
            pl.BlockSpec((hid, mid), lambda i: (0, 0)),
        ],
        out_specs=pl.BlockSpec((BLK, mid), lambda i: (i, 0)),
        out_shape=jax.ShapeDtypeStruct((n_pad, mid), F32),
    )(s_full, t0, dis, W1, b1.reshape(1, -1), W2)


def _tc_final(s_full, t2, dis, b3):
    n_pad, fout = t2.shape
    nb = n_pad // BLK

    def body(s_ref, t_ref, dis_ref, b_ref, out_ref):
        dis = dis_ref[...]
        out_ref[...] = jnp.maximum(dis * (s_ref[...] + t_ref[...]) + b_ref[...],
                                   0.0)

    return pl.pallas_call(
        body,
        grid=(nb,),
        in_specs=[
            pl.BlockSpec((BLK, fout), lambda i: (i, 0)),
            pl.BlockSpec((BLK, fout), lambda i: (i, 0)),
            pl.BlockSpec((BLK, 1), lambda i: (i, 0)),
            pl.BlockSpec((1, fout), lambda i: (0, 0)),
        ],
        out_specs=pl.BlockSpec((BLK, fout), lambda i: (i, 0)),
        out_shape=jax.ShapeDtypeStruct((n_pad, fout), F32),
    )(s_full, t2, dis, b3.reshape(1, -1))


def kernel(x, edge_index, W1, b1, W2, b2, W3, b3):
    n, cin = x.shape
    e = edge_index.shape[1]

    r2 = _cdiv(_cdiv(e, NS * WIN), PIECE) * PIECE
    e_pad = NS * r2 * WIN
    n_pad = (_cdiv(n + 1, BLK)) * BLK

    src = edge_index[0].astype(jnp.int32)
    dst = edge_index[1].astype(jnp.int32)
    pad = jnp.full((e_pad - e,), n, jnp.int32)
    src_p = jnp.concatenate([src, pad])
    dst2 = jnp.concatenate([dst, pad]).reshape(NS, r2, WIN)
    nch = cin // CH
    src8 = (src_p * nch)[None, :] + jnp.arange(nch, dtype=jnp.int32)[:, None]
    src8 = src8.reshape(nch, NS, r2, WIN)
    nch3 = (W3.shape[1]) // CH
    src4 = (src_p * nch3)[None, :] + jnp.arange(nch3, dtype=jnp.int32)[:, None]
    src4 = src4.reshape(nch3, NS, r2, WIN)
    x_p = jnp.pad(x, ((0, n_pad - n), (0, 0)))

    def scat(t, n_chunks, srcb_slabs):
        t8 = t.reshape(n_pad * (t.shape[1] // CH), CH)
        return _sc_scatter(n_chunks, n_pad, r2)(srcb_slabs, dst2, t8)

    hist = _sc_hist(n_pad, r2)(dst2)
    dis, t0 = _tc_prep(hist, x_p, n)
    s0 = scat(t0, nch, src8)
    t1 = _tc_layer1(s0, t0, dis, W1, b1, W2)
    s1 = scat(t1, nch, src8)
    t2 = _tc_layer(s1, t1, dis, b2, W3)
    s2 = scat(t2, nch3, src4)
    out = _tc_final(s2, t2, dis, b3)
    return out[:n]


# speedup vs baseline: 1.7280x; 1.7280x over previous
import functools

import jax
import jax.numpy as jnp
from jax import lax
from jax.experimental import pallas as pl
from jax.experimental.pallas import tpu as pltpu
from jax.experimental.pallas import tpu_sc as plsc

F32 = jnp.float32

NC = 2
NS = 16
WIN = 128
BLK = 512
CH = 32
HCH = 16
NBUF = 4
PIECE = 28


def _cdiv(a, b):
    return (a + b - 1) // b


@functools.lru_cache(maxsize=None)
def _sc_hist(n_pad, r2):
    mesh = plsc.VectorSubcoreMesh(core_axis_name="c", subcore_axis_name="s")
    rpw = n_pad // NS
    zrows = rpw // 4
    rh = r2 // 2

    def body(dst_hbm, out_hbm, dst_v, ones_v, zbuf, acc, sem):
        c = lax.axis_index("c")
        s = lax.axis_index("s")
        pltpu.sync_copy(dst_hbm.at[s, pl.ds(c * rh, rh)], dst_v)

        @pl.loop(0, WIN)
        def _(i):
            ones_v[i, pl.ds(0, HCH)] = jnp.ones((HCH,), F32)

        @pl.loop(0, zrows)
        def _(i):
            zbuf[i, pl.ds(0, HCH)] = jnp.zeros((HCH,), F32)

        for q in range(4):
            pltpu.sync_copy(zbuf, acc.at[pl.ds(s * rpw + q * zrows, zrows)])
        plsc.subcore_barrier()

        @pl.loop(0, rh)
        def _(j):
            pltpu.async_copy(ones_v, acc.at[dst_v.at[j]], sem, add=True).wait()

        plsc.subcore_barrier()
        pltpu.sync_copy(acc.at[pl.ds(s * rpw, rpw)],
                        out_hbm.at[c, pl.ds(s * rpw, rpw)])

    return pl.kernel(
        body,
        mesh=mesh,
        compiler_params=pltpu.CompilerParams(use_tc_tiling_on_sc=False),
        out_type=jax.ShapeDtypeStruct((NC, n_pad, HCH), F32),
        scratch_types=[
            pltpu.VMEM((rh, WIN), jnp.int32),
            pltpu.VMEM((WIN, HCH), F32),
            pltpu.VMEM((zrows, HCH), F32),
            pltpu.VMEM_SHARED((n_pad, HCH), F32),
            pltpu.SemaphoreType.DMA,
        ],
    )


@functools.lru_cache(maxsize=None)
def _sc_scatter(n_chunks, n_pad, r2):
    mesh = plsc.VectorSubcoreMesh(core_axis_name="c", subcore_axis_name="s")
    rpw = n_pad // NS
    zrows = rpw // 32
    cpc = n_chunks // NC
    n_pieces = r2 // PIECE
    assert r2 % PIECE == 0 and PIECE % NBUF == 0

    def body(src8_hbm, dst_hbm, t8_hbm, out_hbm, *rest):
        src_v, dst_v = rest[0], rest[1]
        bufs = rest[2:2 + NBUF]
        zbuf = rest[2 + NBUF]
        acc = rest[3 + NBUF]
        sems_g = rest[4 + NBUF:4 + 2 * NBUF]
        sems_s = rest[4 + 2 * NBUF:4 + 3 * NBUF]
        c = lax.axis_index("c")
        s = lax.axis_index("s")

        @pl.loop(0, zrows)
        def _(i):
            zbuf[i, pl.ds(0, CH)] = jnp.zeros((CH,), F32)

        for ci_l in range(cpc):
            ci = c * cpc + ci_l
            for q in range(32):
                pltpu.sync_copy(zbuf, acc.at[pl.ds(s * rpw + q * zrows, zrows)])
            plsc.subcore_barrier()

            for piece in range(n_pieces):
                pltpu.sync_copy(src8_hbm.at[ci, s, pl.ds(piece * PIECE, PIECE)],
                                src_v)
                pltpu.sync_copy(dst_hbm.at[s, pl.ds(piece * PIECE, PIECE)],
                                dst_v)

                @pl.loop(0, PIECE, step=NBUF)
                def _(j):
                    @pl.when(j > 0)
                    def _():
                        for b in range(NBUF):
                            pltpu.make_async_copy(
                                bufs[b], acc.at[dst_v.at[j + b]],
                                sems_s[b]).wait()
                    gathers = [
                        pltpu.async_copy(t8_hbm.at[src_v.at[j + b]], bufs[b],
                                         sems_g[b])
                        for b in range(NBUF)
                    ]
                    for b in range(NBUF):
                        gathers[b].wait()
                        pltpu.async_copy(bufs[b], acc.at[dst_v.at[j + b]],
                                         sems_s[b], add=True)

                for b in range(NBUF):
                    pltpu.make_async_copy(
                        bufs[b], acc.at[dst_v.at[PIECE - NBUF + b]],
                        sems_s[b]).wait()

            plsc.subcore_barrier()
            pltpu.sync_copy(acc.at[pl.ds(s * rpw, rpw)],
                            out_hbm.at[pl.ds(s * rpw, rpw),
                                       pl.ds(ci * CH, CH)])
            if ci_l + 1 < cpc:
                plsc.subcore_barrier()

    return pl.kernel(
        body,
        mesh=mesh,
        compiler_params=pltpu.CompilerParams(use_tc_tiling_on_sc=False),
        out_type=jax.ShapeDtypeStruct((n_pad, n_chunks * CH), F32),
        scratch_types=[
            pltpu.VMEM((PIECE, WIN), jnp.int32),
            pltpu.VMEM((PIECE, WIN), jnp.int32),
        ] + [pltpu.VMEM((WIN, CH), F32)] * NBUF + [
            pltpu.VMEM((zrows, CH), F32),
            pltpu.VMEM_SHARED((n_pad, CH), F32),
        ] + [pltpu.SemaphoreType.DMA] * (2 * NBUF),
    )


def _dot(a, b):
    return lax.dot_general(a, b, (((1,), (0,)), ((), ())),
                           preferred_element_type=F32)


def _tc_prep(hist, x_p, n_real):
    n_pad, cin = x_p.shape
    nb = n_pad // BLK

    def body(hist_ref, x_ref, dis_ref, t_ref):
        i = pl.program_id(0)
        deg = hist_ref[0] + hist_ref[1] + 1.0
        row = i * BLK + lax.broadcasted_iota(jnp.int32, (BLK, HCH), 0)
        dis = jnp.where(row < n_real, lax.rsqrt(deg), 0.0)
        dis_col = dis[:, 0:1]
        dis_ref[...] = dis_col
        t_ref[...] = x_ref[...] * dis_col

    return pl.pallas_call(
        body,
        grid=(nb,),
        in_specs=[
            pl.BlockSpec((NC, BLK, HCH), lambda i: (0, i, 0)),
            pl.BlockSpec((BLK, cin), lambda i: (i, 0)),
        ],
        out_specs=[pl.BlockSpec((BLK, 1), lambda i: (i, 0)),
                   pl.BlockSpec((BLK, cin), lambda i: (i, 0))],
        out_shape=[jax.ShapeDtypeStruct((n_pad, 1), F32),
                   jax.ShapeDtypeStruct((n_pad, cin), F32)],
    )(hist, x_p)


def _tc_layer(s_full, t_prev, dis, b, W_next):
    n_pad, fin = t_prev.shape
    fout = W_next.shape[1]
    nb = n_pad // BLK

    def body(s_ref, t_ref, dis_ref, b_ref, w_ref, out_ref):
        dis = dis_ref[...]
        h = jnp.maximum(dis * (s_ref[...] + t_ref[...]) + b_ref[...], 0.0)
        out_ref[...] = dis * _dot(h, w_ref[...])

    return pl.pallas_call(
        body,
        grid=(nb,),
        in_specs=[
            pl.BlockSpec((BLK, fin), lambda i: (i, 0)),
            pl.BlockSpec((BLK, fin), lambda i: (i, 0)),
            pl.BlockSpec((BLK, 1), lambda i: (i, 0)),
            pl.BlockSpec((1, fin), lambda i: (0, 0)),
            pl.BlockSpec((fin, fout), lambda i: (0, 0)),
        ],
        out_specs=pl.BlockSpec((BLK, fout), lambda i: (i, 0)),
        out_shape=jax.ShapeDtypeStruct((n_pad, fout), F32),
    )(s_full, t_prev, dis, b.reshape(1, -1), W_next)


def _tc_layer1(s_full, t0, dis, W1, b1, W2):
    n_pad, cin = t0.shape
    hid = W1.shape[1]
    mid = W2.shape[1]
    nb = n_pad // BLK

    def body(s_ref, t_ref, dis_ref, w1_ref, b1_ref, w2_ref, out_ref):
        dis = dis_ref[...]
        p0 = dis * (s_ref[...] + t_ref[...])
        h1 = jnp.maximum(_dot(p0, w1_ref[...]) + b1_ref[...], 0.0)
        out_ref[...] = dis * _dot(h1, w2_ref[...])

    return pl.pallas_call(
        body,
        grid=(nb,),
        in_specs=[
            pl.BlockSpec((BLK, cin), lambda i: (i, 0)),
            pl.BlockSpec((BLK, cin), lambda i: (i, 0)),
            pl.BlockSpec((BLK, 1), lambda i: (i, 0)),
            pl.BlockSpec((cin, hid), lambda i: (0, 0)),
            pl.BlockSpec((1, hid), lambda i: (0, 0)),
            pl.BlockSpec((hid, mid), lambda i: (0, 0)),
        ],
        out_specs=pl.BlockSpec((BLK, mid), lambda i: (i, 0)),
        out_shape=jax.ShapeDtypeStruct((n_pad, mid), F32),
    )(s_full, t0, dis, W1, b1.reshape(1, -1), W2)


def _tc_final(s_full, t2, dis, b3):
    n_pad, fout = t2.shape
    nb = n_pad // BLK

    def body(s_ref, t_ref, dis_ref, b_ref, out_ref):
        dis = dis_ref[...]
        out_ref[...] = jnp.maximum(dis * (s_ref[...] + t_ref[...]) + b_ref[...],
                                   0.0)

    return pl.pallas_call(
        body,
        grid=(nb,),
        in_specs=[
            pl.BlockSpec((BLK, fout), lambda i: (i, 0)),
            pl.BlockSpec((BLK, fout), lambda i: (i, 0)),
            pl.BlockSpec((BLK, 1), lambda i: (i, 0)),
            pl.BlockSpec((1, fout), lambda i: (0, 0)),
        ],
        out_specs=pl.BlockSpec((BLK, fout), lambda i: (i, 0)),
        out_shape=jax.ShapeDtypeStruct((n_pad, fout), F32),
    )(s_full, t2, dis, b3.reshape(1, -1))


def kernel(x, edge_index, W1, b1, W2, b2, W3, b3):
    n, cin = x.shape
    e = edge_index.shape[1]

    r2 = _cdiv(_cdiv(e, NS * WIN), PIECE) * PIECE
    e_pad = NS * r2 * WIN
    n_pad = (_cdiv(n + 1, BLK)) * BLK

    src = edge_index[0].astype(jnp.int32)
    dst = edge_index[1].astype(jnp.int32)
    pad = jnp.full((e_pad - e,), n, jnp.int32)
    src_p = jnp.concatenate([src, pad])
    dst2 = jnp.concatenate([dst, pad]).reshape(NS, r2, WIN)
    nch = cin // CH
    src8 = (src_p * nch)[None, :] + jnp.arange(nch, dtype=jnp.int32)[:, None]
    src8 = src8.reshape(nch, NS, r2, WIN)
    nch3 = (W3.shape[1]) // CH
    src4 = (src_p * nch3)[None, :] + jnp.arange(nch3, dtype=jnp.int32)[:, None]
    src4 = src4.reshape(nch3, NS, r2, WIN)
    x_p = jnp.pad(x, ((0, n_pad - n), (0, 0)))

    def scat(t, n_chunks, srcb_slabs):
        t8 = t.reshape(n_pad * (t.shape[1] // CH), CH)
        return _sc_scatter(n_chunks, n_pad, r2)(srcb_slabs, dst2, t8)

    hist = _sc_hist(n_pad, r2)(dst2)
    dis, t0 = _tc_prep(hist, x_p, n)
    s0 = scat(t0, nch, src8)
    t1 = _tc_layer1(s0, t0, dis, W1, b1, W2)
    s1 = scat(t1, nch, src8)
    t2 = _tc_layer(s1, t1, dis, b2, W3)
    s2 = scat(t2, nch3, src4)
    out = _tc_final(s2, t2, dis, b3)
    return out[:n]
